# baseline (speedup 1.0000x reference)
import functools

import jax
import jax.numpy as jnp
from jax import lax
from jax.experimental import pallas as pl
from jax.experimental.pallas import tpu as pltpu
from jax.experimental.pallas import tpu_sc as plsc

NC = 2
NS = 16
CHUNK = 64
F = 128
NBUF = 4
IB = NBUF


def _sc_segment_sum(n_pad, nb, feat, idx5, zagg, zdeg, ones16):
    rows_per_tile = n_pad // NS

    mesh = plsc.VectorSubcoreMesh(
        core_axis_name="c", subcore_axis_name="s",
        num_cores=NC, num_subcores=NS)

    @functools.partial(
        pl.kernel,
        out_type=[
            jax.ShapeDtypeStruct((NC, n_pad, F), jnp.float32),
            jax.ShapeDtypeStruct((NC, n_pad, 16), jnp.float32),
        ],
        mesh=mesh,
        scratch_types=[
            pltpu.VMEM((IB, 2, CHUNK), jnp.int32),
            pltpu.VMEM((IB, 2, CHUNK), jnp.int32),
            pltpu.VMEM((NBUF, CHUNK, F), jnp.float32),
            pltpu.VMEM((CHUNK, 16), jnp.float32),
            pltpu.VMEM_SHARED((n_pad, F), jnp.float32),
            pltpu.VMEM_SHARED((n_pad, 16), jnp.float32),
            pltpu.SemaphoreType.DMA,
            pltpu.SemaphoreType.DMA,
            pltpu.SemaphoreType.DMA,
            pltpu.SemaphoreType.DMA,
            pltpu.SemaphoreType.DMA,
            pltpu.SemaphoreType.DMA,
            pltpu.SemaphoreType.DMA,
            pltpu.SemaphoreType.DMA,
        ],
        compiler_params=pltpu.CompilerParams(use_tc_tiling_on_sc=False),
    )
    def sc_fn(feat_hbm, idx_hbm,
              zagg_hbm, zdeg_hbm, o16_hbm,
              agg_out, deg_out,
              ring0, ring1, rowbuf, onesbuf,
              agg_sp, deg_sp,
              gsem0, gsem1, gsem2, gsem3, ssem, osem, isem0, isem1):
        c = lax.axis_index("c")
        s = lax.axis_index("s")
        base = s * rows_per_tile
        gsems = (gsem0, gsem1, gsem2, gsem3)

        pltpu.sync_copy(o16_hbm, onesbuf)
        pltpu.async_copy(idx_hbm.at[c, s, 0], ring0, isem0)
        pltpu.async_copy(idx_hbm.at[c, s, 1], ring1, isem1)
        pltpu.sync_copy(zagg_hbm, agg_sp.at[pl.ds(base, rows_per_tile)])
        pltpu.sync_copy(zdeg_hbm, deg_sp.at[pl.ds(base, rows_per_tile)])
        plsc.subcore_barrier()

        pltpu.make_async_copy(idx_hbm.at[c, s, 0], ring0, isem0).wait()
        for jj in range(NBUF):
            pltpu.async_copy(feat_hbm.at[ring0.at[jj, 0]],
                             rowbuf.at[jj], gsems[jj])

        def block(ib, ring, nring, isem, nisem):
            for jj in range(IB):
                buf, gsem = rowbuf.at[jj], gsems[jj]
                pltpu.make_async_copy(feat_hbm.at[ring.at[jj, 0]],
                                      buf, gsem).wait()
                pltpu.async_copy(buf, agg_sp.at[ring.at[jj, 1]], ssem,
                                 add=True)
                pltpu.async_copy(onesbuf, deg_sp.at[ring.at[jj, 1]], osem,
                                 add=True)
                pltpu.make_async_copy(buf, agg_sp.at[ring.at[jj, 1]],
                                      ssem).wait()

                @pl.when(ib + 1 < nb)
                def _():
                    if jj == 0:
                        pltpu.make_async_copy(idx_hbm.at[c, s, ib + 1],
                                              nring, nisem).wait()
                    pltpu.async_copy(feat_hbm.at[nring.at[jj, 0]],
                                     buf, gsem)

            @pl.when(ib + 2 < nb)
            def _():
                pltpu.async_copy(idx_hbm.at[c, s, ib + 2], ring, isem)

        def body(g, carry):
            ib = g * 2
            block(ib, ring0, ring1, isem0, isem1)
            block(ib + 1, ring1, ring0, isem1, isem0)
            return carry

        lax.fori_loop(0, nb // 2, body, 0)

        def drain(j, carry):
            pltpu.make_async_copy(onesbuf, deg_sp.at[ring0.at[0, 1]],
                                  osem).wait()
            return carry

        lax.fori_loop(0, nb * IB, drain, 0)

        plsc.subcore_barrier()

        pltpu.sync_copy(agg_sp.at[pl.ds(base, rows_per_tile)],
                        agg_out.at[c, pl.ds(base, rows_per_tile)])
        pltpu.sync_copy(deg_sp.at[pl.ds(base, rows_per_tile)],
                        deg_out.at[c, pl.ds(base, rows_per_tile)])

    return sc_fn(feat, idx5, zagg, zdeg, ones16)


def _tc_epilogue(n, n_pad, agg0, agg1, deg0, deg1, W, b2d):
    blk = 1024
    dn = (((1,), (1,)), ((), ()))

    def body(a0, a1, d0, d1, w, bv, o):
        deg = d0[...][:, 0:1] + d1[...][:, 0:1]
        o[...] = (
            lax.dot_general(a0[...] + a1[...], w[...], dn,
                            preferred_element_type=jnp.float32)
            + deg * bv[...])

    return pl.pallas_call(
        body,
        grid=(n_pad // blk,),
        in_specs=[
            pl.BlockSpec((blk, F), lambda i: (i, 0)),
            pl.BlockSpec((blk, F), lambda i: (i, 0)),
            pl.BlockSpec((blk, 16), lambda i: (i, 0)),
            pl.BlockSpec((blk, 16), lambda i: (i, 0)),
            pl.BlockSpec((128, F), lambda i: (0, 0)),
            pl.BlockSpec((1, 128), lambda i: (0, 0)),
        ],
        out_specs=pl.BlockSpec((blk, 128), lambda i: (i, 0)),
        out_shape=jax.ShapeDtypeStruct((n, 128), jnp.float32),
    )(agg0, agg1, deg0, deg1, W, b2d)


def kernel(feat, edge_index, W, b):
    n = feat.shape[0]
    e = edge_index.shape[1]
    n_pad = ((n + 2047) // 2048) * 2048
    gsz = 2 * IB * CHUNK
    epw = gsz * (-(-e // (NC * NS * gsz)))
    e_pad = NC * NS * epw
    nb = epw // (IB * CHUNK)

    src = edge_index[0].astype(jnp.int32)
    dst = edge_index[1].astype(jnp.int32)
    pad = jnp.arange(e_pad - e, dtype=jnp.int32)
    src5 = jnp.concatenate([src, pad % n]).reshape(NC, NS, nb, IB, 1, CHUNK)
    dst5 = jnp.concatenate(
        [dst, n + pad % (n_pad - n)]).reshape(NC, NS, nb, IB, 1, CHUNK)
    idx5 = jnp.concatenate([src5, dst5], axis=4)

    zagg = jnp.zeros((n_pad // NS, F), jnp.float32)
    zdeg = jnp.zeros((n_pad // NS, 16), jnp.float32)
    ones16 = jnp.ones((CHUNK, 16), jnp.float32)

    agg, deg = _sc_segment_sum(n_pad, nb, feat, idx5, zagg, zdeg, ones16)
    return _tc_epilogue(n, n_pad, agg[0], agg[1], deg[0], deg[1],
                        W, b.reshape(1, -1))


# speedup vs baseline: 1.0696x; 1.0696x over previous
import functools

import jax
import jax.numpy as jnp
from jax import lax
from jax.experimental import pallas as pl
from jax.experimental.pallas import tpu as pltpu
from jax.experimental.pallas import tpu_sc as plsc

NC = 2
NS = 16
CHUNK = 128
F = 128
IB = 4


def _sc_segment_sum(n_pad, nb, feat, idx5, zagg, zdeg, ones16):
    rows_per_tile = n_pad // NS

    mesh = plsc.VectorSubcoreMesh(
        core_axis_name="c", subcore_axis_name="s",
        num_cores=NC, num_subcores=NS)

    @functools.partial(
        pl.kernel,
        out_type=[
            jax.ShapeDtypeStruct((NC, n_pad, F), jnp.float32),
            jax.ShapeDtypeStruct((NC, n_pad, 16), jnp.float32),
        ],
        mesh=mesh,
        scratch_types=[
            pltpu.VMEM((IB, 2, CHUNK), jnp.int32),
            pltpu.VMEM((IB, 2, CHUNK), jnp.int32),
            pltpu.VMEM((CHUNK, F), jnp.float32),
            pltpu.VMEM((CHUNK, F), jnp.float32),
            pltpu.VMEM((CHUNK, 16), jnp.float32),
            pltpu.VMEM_SHARED((n_pad, F), jnp.float32),
            pltpu.VMEM_SHARED((n_pad, 16), jnp.float32),
            pltpu.SemaphoreType.DMA,
            pltpu.SemaphoreType.DMA,
            pltpu.SemaphoreType.DMA,
            pltpu.SemaphoreType.DMA,
            pltpu.SemaphoreType.DMA,
            pltpu.SemaphoreType.DMA,
            pltpu.SemaphoreType.DMA,
        ],
        compiler_params=pltpu.CompilerParams(use_tc_tiling_on_sc=False),
    )
    def sc_fn(feat_hbm, idx_hbm,
              zagg_hbm, zdeg_hbm, o16_hbm,
              agg_out, deg_out,
              ring0, ring1, rowbuf0, rowbuf1, onesbuf,
              agg_sp, deg_sp, gsem0, gsem1, ssem, osem, isem0, isem1, zsem):
        c = lax.axis_index("c")
        s = lax.axis_index("s")
        base = s * rows_per_tile

        pltpu.sync_copy(o16_hbm, onesbuf)
        pltpu.async_copy(idx_hbm.at[c, s, 0], ring0, isem0)
        pltpu.async_copy(idx_hbm.at[c, s, 1], ring1, isem1)
        pltpu.make_async_copy(idx_hbm.at[c, s, 0], ring0, isem0).wait()
        pltpu.async_copy(feat_hbm.at[ring0.at[0, 0]], rowbuf0, gsem0)
        pltpu.async_copy(feat_hbm.at[ring0.at[1, 0]], rowbuf1, gsem1)
        pltpu.async_copy(zagg_hbm, agg_sp.at[pl.ds(base, rows_per_tile)],
                         zsem)
        pltpu.async_copy(zdeg_hbm, deg_sp.at[pl.ds(base, rows_per_tile)],
                         zsem)
        pltpu.make_async_copy(zagg_hbm, agg_sp.at[pl.ds(base, rows_per_tile)],
                              zsem).wait()
        pltpu.make_async_copy(zdeg_hbm, deg_sp.at[pl.ds(base, rows_per_tile)],
                              zsem).wait()
        plsc.subcore_barrier()

        def block(ib, ring, nring, isem, nisem):
            for jj in range(IB):
                buf, gsem = (rowbuf0, gsem0) if jj % 2 == 0 else (rowbuf1,
                                                                  gsem1)
                pltpu.make_async_copy(feat_hbm.at[ring.at[jj, 0]],
                                      buf, gsem).wait()
                pltpu.async_copy(buf, agg_sp.at[ring.at[jj, 1]], ssem,
                                 add=True)
                pltpu.async_copy(onesbuf, deg_sp.at[ring.at[jj, 1]], osem,
                                 add=True)
                pltpu.make_async_copy(buf, agg_sp.at[ring.at[jj, 1]],
                                      ssem).wait()
                if jj + 2 < IB:
                    pltpu.async_copy(feat_hbm.at[ring.at[jj + 2, 0]],
                                     buf, gsem)
                elif jj + 2 == IB:
                    @pl.when(ib + 1 < nb)
                    def _():
                        pltpu.make_async_copy(idx_hbm.at[c, s, ib + 1],
                                              nring, nisem).wait()
                        pltpu.async_copy(feat_hbm.at[nring.at[0, 0]],
                                         buf, gsem)
                else:
                    @pl.when(ib + 1 < nb)
                    def _():
                        pltpu.async_copy(feat_hbm.at[nring.at[1, 0]],
                                         buf, gsem)

            @pl.when(ib + 2 < nb)
            def _():
                pltpu.async_copy(idx_hbm.at[c, s, ib + 2], ring, isem)

        def body(g, carry):
            ib = g * 2
            block(ib, ring0, ring1, isem0, isem1)
            block(ib + 1, ring1, ring0, isem1, isem0)
            return carry

        lax.fori_loop(0, nb // 2, body, 0)

        def drain(j, carry):
            pltpu.make_async_copy(onesbuf, deg_sp.at[ring0.at[0, 1]],
                                  osem).wait()
            return carry

        lax.fori_loop(0, nb * IB, drain, 0)

        plsc.subcore_barrier()

        pltpu.async_copy(agg_sp.at[pl.ds(base, rows_per_tile)],
                         agg_out.at[c, pl.ds(base, rows_per_tile)], zsem)
        pltpu.async_copy(deg_sp.at[pl.ds(base, rows_per_tile)],
                         deg_out.at[c, pl.ds(base, rows_per_tile)], zsem)
        pltpu.make_async_copy(agg_sp.at[pl.ds(base, rows_per_tile)],
                              agg_out.at[c, pl.ds(base, rows_per_tile)],
                              zsem).wait()
        pltpu.make_async_copy(deg_sp.at[pl.ds(base, rows_per_tile)],
                              deg_out.at[c, pl.ds(base, rows_per_tile)],
                              zsem).wait()

    return sc_fn(feat, idx5, zagg, zdeg, ones16)


def _tc_epilogue(n, n_pad, agg0, agg1, deg0, deg1, W, b2d):
    blk = 1024
    dn = (((1,), (1,)), ((), ()))

    def body(a0, a1, d0, d1, w, bv, o):
        deg = d0[...][:, 0:1] + d1[...][:, 0:1]
        o[...] = (
            lax.dot_general(a0[...] + a1[...], w[...], dn,
                            preferred_element_type=jnp.float32)
            + deg * bv[...])

    return pl.pallas_call(
        body,
        grid=(n_pad // blk,),
        in_specs=[
            pl.BlockSpec((blk, F), lambda i: (i, 0)),
            pl.BlockSpec((blk, F), lambda i: (i, 0)),
            pl.BlockSpec((blk, 16), lambda i: (i, 0)),
            pl.BlockSpec((blk, 16), lambda i: (i, 0)),
            pl.BlockSpec((128, F), lambda i: (0, 0)),
            pl.BlockSpec((1, 128), lambda i: (0, 0)),
        ],
        out_specs=pl.BlockSpec((blk, 128), lambda i: (i, 0)),
        out_shape=jax.ShapeDtypeStruct((n, 128), jnp.float32),
    )(agg0, agg1, deg0, deg1, W, b2d)


def kernel(feat, edge_index, W, b):
    n = feat.shape[0]
    e = edge_index.shape[1]
    n_pad = ((n + 2047) // 2048) * 2048
    gsz = 2 * IB * CHUNK
    epw = gsz * (-(-e // (NC * NS * gsz)))
    e_pad = NC * NS * epw
    nb = epw // (IB * CHUNK)

    src = edge_index[0].astype(jnp.int32)
    dst = edge_index[1].astype(jnp.int32)
    pad = jnp.arange(e_pad - e, dtype=jnp.int32)
    src5 = jnp.concatenate([src, pad % n]).reshape(NC, NS, nb, IB, 1, CHUNK)
    dst5 = jnp.concatenate(
        [dst, n + pad % (n_pad - n)]).reshape(NC, NS, nb, IB, 1, CHUNK)
    idx5 = jnp.concatenate([src5, dst5], axis=4)

    zagg = jnp.zeros((n_pad // NS, F), jnp.float32)
    zdeg = jnp.zeros((n_pad // NS, 16), jnp.float32)
    ones16 = jnp.ones((CHUNK, 16), jnp.float32)

    agg, deg = _sc_segment_sum(n_pad, nb, feat, idx5, zagg, zdeg, ones16)
    return _tc_epilogue(n, n_pad, agg[0], agg[1], deg[0], deg[1],
                        W, b.reshape(1, -1))
